# Initial kernel scaffold; baseline (speedup 1.0000x reference)
import jax
import jax.numpy as jnp
from jax.experimental import pallas as pl


def kernel(context_words, target_words, negative_samples, emb_in, emb_out):
    raise NotImplementedError("write your pallas kernel here")


# speedup vs baseline: 3.3970x; 3.3970x over previous
import functools

import jax
import jax.numpy as jnp
from jax import lax
from jax.experimental import pallas as pl
from jax.experimental.pallas import tpu as pltpu
from jax.experimental.pallas import tpu_sc as plsc

NC = 2
NS = 16
NW = NC * NS
LANES = 16
IW = 80


def _sc_scores(ctx_idx, tgt_idx, neg_idx, emb_in, emb_out, B, CTX, NEG, D):
    bpw = B // NW
    CB = 16
    nch = bpw // CB
    nseg = D // LANES
    cgrp = CB * CTX // IW
    ngrp = CB * NEG // IW
    inv_ctx = jnp.float32(1.0 / CTX)

    mesh = plsc.VectorSubcoreMesh(core_axis_name="c", subcore_axis_name="s")

    @functools.partial(
        pl.kernel,
        mesh=mesh,
        compiler_params=pltpu.CompilerParams(
            needs_layout_passes=False, use_tc_tiling_on_sc=False),
        out_type=jax.ShapeDtypeStruct((B, 32), jnp.float32),
        scratch_types=[
            pltpu.VMEM((2, cgrp, IW), jnp.int32),
            pltpu.VMEM((2, ngrp, IW), jnp.int32),
            pltpu.VMEM((2, 1, CB), jnp.int32),
            pltpu.VMEM((2, CB * CTX, D), jnp.float32),
            pltpu.VMEM((2, CB * NEG, D), jnp.float32),
            pltpu.VMEM((2, CB, D), jnp.float32),
            pltpu.VMEM((CB, 32), jnp.float32),
            pltpu.SemaphoreType.DMA,
            pltpu.SemaphoreType.DMA,
        ],
    )
    def sc_kernel(ctx_i_hbm, tgt_i_hbm, neg_i_hbm, ein_hbm, eout_hbm, out_hbm,
                  ctxi_v, negi_v, tgti_v, ctxr_v, negr_v, tgtr_v, sc_v,
                  sem0, sem1):
        wid = lax.axis_index("s") * NC + lax.axis_index("c")
        sems = (sem0, sem1)

        def issue(c, p):
            rc = wid * (nch * cgrp) + c * cgrp
            rn = wid * (nch * ngrp) + c * ngrp
            pltpu.sync_copy(ctx_i_hbm.at[pl.ds(rc, cgrp)], ctxi_v.at[p])
            pltpu.sync_copy(neg_i_hbm.at[pl.ds(rn, ngrp)], negi_v.at[p])
            pltpu.sync_copy(tgt_i_hbm.at[pl.ds(wid * nch + c, 1)], tgti_v.at[p])
            sem = sems[p]
            for j in range(cgrp):
                pltpu.async_copy(ein_hbm.at[ctxi_v.at[p, j]],
                                 ctxr_v.at[p, pl.ds(j * IW, IW)], sem)
            for j in range(ngrp):
                pltpu.async_copy(eout_hbm.at[negi_v.at[p, j]],
                                 negr_v.at[p, pl.ds(j * IW, IW)], sem)
            pltpu.async_copy(eout_hbm.at[tgti_v.at[p, 0]], tgtr_v.at[p], sem)

        def drain(p):
            sem = sems[p]
            for j in range(cgrp):
                pltpu.make_async_copy(ein_hbm.at[ctxi_v.at[p, j]],
                                      ctxr_v.at[p, pl.ds(j * IW, IW)],
                                      sem).wait()
            for j in range(ngrp):
                pltpu.make_async_copy(eout_hbm.at[negi_v.at[p, j]],
                                      negr_v.at[p, pl.ds(j * IW, IW)],
                                      sem).wait()
            pltpu.make_async_copy(eout_hbm.at[tgti_v.at[p, 0]], tgtr_v.at[p],
                                  sem).wait()

        lane = lax.iota(jnp.int32, LANES)
        crow = [lane * CTX + j for j in range(CTX)]
        nrow = [lane * NEG + k for k in range(NEG)]
        zero = jnp.zeros((LANES,), jnp.float32)

        def compute(c, p):
            def dbody(d, carry):
                pos = carry[0]
                negs = carry[1:]
                dcol = jnp.broadcast_to(d, (LANES,))
                acc = plsc.load_gather(ctxr_v.at[p], [crow[0], dcol])
                for j in range(1, CTX):
                    acc = acc + plsc.load_gather(ctxr_v.at[p], [crow[j], dcol])
                acc = acc * inv_ctx
                tcol = plsc.load_gather(tgtr_v.at[p], [lane, dcol])
                pos = pos + acc * tcol
                negs = [n + acc * plsc.load_gather(negr_v.at[p], [nrow[k], dcol])
                        for k, n in enumerate(negs)]
                return [pos] + negs

            res = lax.fori_loop(0, D, dbody, [zero] * (NEG + 1))
            plsc.store_scatter(sc_v, [lane, jnp.broadcast_to(0, (LANES,))],
                               -res[0])
            for k in range(NEG):
                plsc.store_scatter(sc_v,
                                   [lane, jnp.broadcast_to(k + 1, (LANES,))],
                                   res[k + 1])
            pltpu.sync_copy(sc_v, out_hbm.at[pl.ds(wid * bpw + c * CB, CB)])

        for z in range(CB):
            sc_v[z, pl.ds(16, 16)] = jnp.zeros((LANES,), jnp.float32)

        issue(0, 0)

        def pair(i, carry):
            for pp in range(2):
                c = i * 2 + pp

                @pl.when(c + 1 < nch)
                def _():
                    issue(c + 1, 1 - pp)

                drain(pp)
                compute(c, pp)
            return carry

        lax.fori_loop(0, nch // 2, pair, 0)

    return sc_kernel(ctx_idx, tgt_idx, neg_idx, emb_in, emb_out)


def _tc_loss(scores, B, NEG):
    RB = 2048
    grid = B // RB

    def body(s_ref, o_ref):
        i = pl.program_id(0)
        x = s_ref[...]
        col = lax.broadcasted_iota(jnp.int32, x.shape, 1)
        sp = jnp.maximum(x, 0.0) + jnp.log1p(jnp.exp(-jnp.abs(x)))
        sp = jnp.where(col < NEG + 1, sp, 0.0)
        part = jnp.sum(sp)

        @pl.when(i == 0)
        def _():
            o_ref[0, 0] = 0.0

        o_ref[0, 0] += part

        @pl.when(i == grid - 1)
        def _():
            o_ref[0, 0] = o_ref[0, 0] * jnp.float32(1.0 / B)

    return pl.pallas_call(
        body,
        grid=(grid,),
        in_specs=[pl.BlockSpec((RB, 32), lambda i: (i, 0))],
        out_specs=pl.BlockSpec(memory_space=pltpu.SMEM),
        out_shape=jax.ShapeDtypeStruct((1, 1), jnp.float32),
    )(scores)


def kernel(context_words, target_words, negative_samples, emb_in, emb_out):
    B, CTX = context_words.shape
    NEG = negative_samples.shape[1]
    D = emb_in.shape[1]

    ctx_idx = context_words.astype(jnp.int32).reshape(B * CTX // IW, IW)
    neg_idx = negative_samples.astype(jnp.int32).reshape(B * NEG // IW, IW)
    tgt_idx = target_words.astype(jnp.int32).reshape(B // 16, 16)

    scores = _sc_scores(ctx_idx, tgt_idx, neg_idx, emb_in, emb_out,
                        B, CTX, NEG, D)
    loss = _tc_loss(scores, B, NEG)
    return loss[0, 0]


# speedup vs baseline: 4.1206x; 1.2130x over previous
import functools

import jax
import jax.numpy as jnp
from jax import lax
from jax.experimental import pallas as pl
from jax.experimental.pallas import tpu as pltpu
from jax.experimental.pallas import tpu_sc as plsc

NC = 2
NS = 16
NW = NC * NS
LANES = 16
IW = 80


def _sc_scores(ctx_idx, tgt_idx, neg_idx, emb_in, emb_out, B, CTX, NEG, D):
    bpw = B // NW
    CB = 16
    nch = bpw // CB
    nseg = D // LANES
    cgrp = CB * CTX // IW
    ngrp = CB * NEG // IW
    inv_ctx = jnp.float32(1.0 / CTX)

    mesh = plsc.VectorSubcoreMesh(core_axis_name="c", subcore_axis_name="s")

    @functools.partial(
        pl.kernel,
        mesh=mesh,
        compiler_params=pltpu.CompilerParams(
            needs_layout_passes=False, use_tc_tiling_on_sc=False),
        out_type=jax.ShapeDtypeStruct((B, 32), jnp.float32),
        scratch_types=[
            pltpu.VMEM((2, cgrp, IW), jnp.int32),
            pltpu.VMEM((2, ngrp, IW), jnp.int32),
            pltpu.VMEM((2, 1, CB), jnp.int32),
            pltpu.VMEM((2, CB * CTX, D), jnp.float32),
            pltpu.VMEM((2, CB * NEG, D), jnp.float32),
            pltpu.VMEM((2, CB, D), jnp.float32),
            pltpu.VMEM((CB, D), jnp.float32),
            pltpu.VMEM((CB, 32), jnp.float32),
            pltpu.SemaphoreType.DMA,
            pltpu.SemaphoreType.DMA,
        ],
    )
    def sc_kernel(ctx_i_hbm, tgt_i_hbm, neg_i_hbm, ein_hbm, eout_hbm, out_hbm,
                  ctxi_v, negi_v, tgti_v, ctxr_v, negr_v, tgtr_v, ctxv_v,
                  sc_v, sem0, sem1):
        wid = lax.axis_index("s") * NC + lax.axis_index("c")
        sems = (sem0, sem1)

        def issue(c, p):
            rc = wid * (nch * cgrp) + c * cgrp
            rn = wid * (nch * ngrp) + c * ngrp
            pltpu.sync_copy(ctx_i_hbm.at[pl.ds(rc, cgrp)], ctxi_v.at[p])
            pltpu.sync_copy(neg_i_hbm.at[pl.ds(rn, ngrp)], negi_v.at[p])
            pltpu.sync_copy(tgt_i_hbm.at[pl.ds(wid * nch + c, 1)], tgti_v.at[p])
            sem = sems[p]
            for j in range(cgrp):
                pltpu.async_copy(ein_hbm.at[ctxi_v.at[p, j]],
                                 ctxr_v.at[p, pl.ds(j * IW, IW)], sem)
            for j in range(ngrp):
                pltpu.async_copy(eout_hbm.at[negi_v.at[p, j]],
                                 negr_v.at[p, pl.ds(j * IW, IW)], sem)
            pltpu.async_copy(eout_hbm.at[tgti_v.at[p, 0]], tgtr_v.at[p], sem)

        def drain(p):
            sem = sems[p]
            for j in range(cgrp):
                pltpu.make_async_copy(ein_hbm.at[ctxi_v.at[p, j]],
                                      ctxr_v.at[p, pl.ds(j * IW, IW)],
                                      sem).wait()
            for j in range(ngrp):
                pltpu.make_async_copy(eout_hbm.at[negi_v.at[p, j]],
                                      negr_v.at[p, pl.ds(j * IW, IW)],
                                      sem).wait()
            pltpu.make_async_copy(eout_hbm.at[tgti_v.at[p, 0]], tgtr_v.at[p],
                                  sem).wait()

        lane = lax.iota(jnp.int32, LANES)
        lane_neg = lane * NEG
        zero = jnp.zeros((LANES,), jnp.float32)

        def compute(c, p):
            def bbody(b, carry):
                base = b * CTX
                for s in range(nseg):
                    v = [ctxr_v[p, base + j, pl.ds(s * LANES, LANES)]
                         for j in range(CTX)]
                    while len(v) > 1:
                        v = [v[i] + v[i + 1] for i in range(0, len(v) - 1, 2)] \
                            + ([v[-1]] if len(v) % 2 else [])
                    ctxv_v[b, pl.ds(s * LANES, LANES)] = v[0] * inv_ctx
                return carry

            lax.fori_loop(0, CB, bbody, 0, unroll=2)

            def dbody(d, carry):
                pos = carry[0]
                negs = carry[1:]
                dcol = jnp.broadcast_to(d, (LANES,))
                acc = plsc.load_gather(ctxv_v, [lane, dcol])
                tcol = plsc.load_gather(tgtr_v.at[p], [lane, dcol])
                pos = pos + acc * tcol
                negs = [n + acc * plsc.load_gather(negr_v.at[p],
                                                   [lane_neg + k, dcol])
                        for k, n in enumerate(negs)]
                return [pos] + negs

            res = lax.fori_loop(0, D, dbody, [zero] * (NEG + 1), unroll=2)
            plsc.store_scatter(sc_v, [lane, jnp.broadcast_to(0, (LANES,))],
                               -res[0])
            for k in range(NEG):
                plsc.store_scatter(sc_v,
                                   [lane, jnp.broadcast_to(k + 1, (LANES,))],
                                   res[k + 1])
            pltpu.sync_copy(sc_v, out_hbm.at[pl.ds(wid * bpw + c * CB, CB)])

        for z in range(CB):
            sc_v[z, pl.ds(16, 16)] = jnp.zeros((LANES,), jnp.float32)

        issue(0, 0)

        def pair(i, carry):
            for pp in range(2):
                c = i * 2 + pp

                @pl.when(c + 1 < nch)
                def _():
                    issue(c + 1, 1 - pp)

                drain(pp)
                compute(c, pp)
            return carry

        lax.fori_loop(0, nch // 2, pair, 0)

    return sc_kernel(ctx_idx, tgt_idx, neg_idx, emb_in, emb_out)


def _tc_loss(scores, B, NEG):
    RB = 2048
    grid = B // RB

    def body(s_ref, o_ref):
        i = pl.program_id(0)
        x = s_ref[...]
        col = lax.broadcasted_iota(jnp.int32, x.shape, 1)
        sp = jnp.maximum(x, 0.0) + jnp.log1p(jnp.exp(-jnp.abs(x)))
        sp = jnp.where(col < NEG + 1, sp, 0.0)
        part = jnp.sum(sp)

        @pl.when(i == 0)
        def _():
            o_ref[0, 0] = 0.0

        o_ref[0, 0] += part

        @pl.when(i == grid - 1)
        def _():
            o_ref[0, 0] = o_ref[0, 0] * jnp.float32(1.0 / B)

    return pl.pallas_call(
        body,
        grid=(grid,),
        in_specs=[pl.BlockSpec((RB, 32), lambda i: (i, 0))],
        out_specs=pl.BlockSpec(memory_space=pltpu.SMEM),
        out_shape=jax.ShapeDtypeStruct((1, 1), jnp.float32),
    )(scores)


def kernel(context_words, target_words, negative_samples, emb_in, emb_out):
    B, CTX = context_words.shape
    NEG = negative_samples.shape[1]
    D = emb_in.shape[1]

    ctx_idx = context_words.astype(jnp.int32).reshape(B * CTX // IW, IW)
    neg_idx = negative_samples.astype(jnp.int32).reshape(B * NEG // IW, IW)
    tgt_idx = target_words.astype(jnp.int32).reshape(B // 16, 16)

    scores = _sc_scores(ctx_idx, tgt_idx, neg_idx, emb_in, emb_out,
                        B, CTX, NEG, D)
    loss = _tc_loss(scores, B, NEG)
    return loss[0, 0]
